# baseline (speedup 1.0000x reference)
import functools

import jax
import jax.numpy as jnp
from jax import lax
from jax.experimental import pallas as pl
from jax.experimental.pallas import tpu as pltpu
from jax.experimental.pallas import tpu_sc as plsc

N = 10000
E = 320000
D = 128
G = 64

CH = 64
TPC = 159
EPT = TPC * CH
DEAD = 10016
RPT = 640
NPAD = 10240

BLK = 1000
NB = N // BLK

_HI = jax.lax.Precision.HIGHEST


NCPAD = 10240


def _make_sc_spmm(with_cnt):
  out_type = [jax.ShapeDtypeStruct((2, NPAD, D), jnp.float32)]
  scratch = [
      pltpu.VMEM((TPC, CH), jnp.int32),
      [pltpu.VMEM((CH,), jnp.int32) for _ in range(3)],
      [pltpu.VMEM((CH,), jnp.int32) for _ in range(3)],
      [pltpu.VMEM((CH, D), jnp.float32) for _ in range(3)],
      pltpu.VMEM_SHARED((NPAD, D), jnp.float32),
      [pltpu.SemaphoreType.DMA for _ in range(3)],
      [pltpu.SemaphoreType.DMA for _ in range(3)],
  ]
  if with_cnt:
    out_type.append(jax.ShapeDtypeStruct((2, NCPAD), jnp.float32))
    scratch += [
        pltpu.VMEM((CH,), jnp.float32),
        pltpu.VMEM_SHARED((NCPAD,), jnp.float32),
    ]

  mesh = plsc.VectorSubcoreMesh(core_axis_name="c", subcore_axis_name="s")

  def body(x_hbm, src_hbm, zeros_hbm, ones_hbm, *rest):
    if with_cnt:
      (acc_out, cnt_out, pk_v, ss, dd, rows, acc_sh, gsem, ssem,
       ones_v, cnt_sh) = rest
    else:
      (acc_out, pk_v, ss, dd, rows, acc_sh, gsem, ssem) = rest
    c = lax.axis_index("c")
    s = lax.axis_index("s")
    wid = c * 16 + s

    pltpu.sync_copy(zeros_hbm, acc_sh.at[pl.ds(s * RPT, RPT)])
    if with_cnt:
      pltpu.sync_copy(ones_hbm, ones_v)

      def zloop(j, _):
        idx = j * 16 + s
        pltpu.sync_copy(zeros_hbm.at[0], cnt_sh.at[pl.ds(idx * D, D)])
        return 0
      lax.fori_loop(0, NCPAD // (16 * D), zloop, 0)

    pltpu.sync_copy(src_hbm.at[wid], pk_v)
    plsc.subcore_barrier()

    def _unpack(j, k):
      for q in range(CH // 16):
        pk = pk_v[j, pl.ds(q * 16, 16)]
        ss[k][pl.ds(q * 16, 16)] = pk & 0xFFFF
        dd[k][pl.ds(q * 16, 16)] = lax.shift_right_logical(pk, 16)

    def _issue_gather(k):
      pltpu.async_copy(x_hbm.at[ss[k]], rows[k], gsem[k])

    def _wait_gather(k):
      pltpu.make_async_copy(x_hbm.at[ss[k]], rows[k], gsem[k]).wait()

    def _issue_scatter(k):
      pltpu.async_copy(rows[k], acc_sh.at[dd[k]], ssem[k], add=True)
      if with_cnt:
        pltpu.async_copy(ones_v, cnt_sh.at[dd[k]], ssem[k], add=True)

    def _wait_scatter(k):
      pltpu.make_async_copy(rows[k], acc_sh.at[dd[k]], ssem[k]).wait()
      if with_cnt:
        pltpu.make_async_copy(ones_v, cnt_sh.at[dd[k]], ssem[k]).wait()

    for k in range(3):
      _unpack(k, k)
      _issue_gather(k)

    def chunk3(t, _):
      base = 3 * t
      for k in range(3):
        _wait_gather(k)
        _issue_scatter(k)
      for k in range(3):
        _wait_scatter(k)
        _unpack(base + 3 + k, k)
        _issue_gather(k)
      return 0
    lax.fori_loop(0, TPC // 3 - 1, chunk3, 0)

    for k in range(3):
      _wait_gather(k)
      _issue_scatter(k)
    for k in range(3):
      _wait_scatter(k)

    plsc.subcore_barrier()
    pltpu.sync_copy(acc_sh.at[pl.ds(s * RPT, RPT)],
                    acc_out.at[c, pl.ds(s * RPT, RPT)])
    if with_cnt:
      @pl.when(s == 0)
      def _():
        pltpu.sync_copy(cnt_sh, cnt_out.at[c])

  return functools.partial(pl.kernel, out_type=out_type, mesh=mesh,
                           scratch_types=scratch)(body)


_sc_spmm_cnt = _make_sc_spmm(True)
_sc_spmm = _make_sc_spmm(False)


def _tc1_body(aggp, cnt0, cnt1, x, batch, wrel, wroot, b,
              h_ref, pool_ref, pool_acc, gcnt_acc):
  i = pl.program_id(0)

  @pl.when(i == 0)
  def _():
    pool_acc[...] = jnp.zeros_like(pool_acc)
    gcnt_acc[...] = jnp.zeros_like(gcnt_acc)

  cnt = cnt0[0, 0, :] + cnt1[0, 0, :]
  inv = 1.0 / jnp.maximum(cnt, 1.0)
  a = aggp[...]
  agg = (a[0] + a[1]) * inv[:, None]
  h = (jnp.dot(agg, wrel[...], precision=_HI)
       + jnp.dot(x[...], wroot[...], precision=_HI) + b[...])
  h = jnp.maximum(h, 0.0)
  h_ref[...] = h

  bt = batch[0, 0, :]
  oh = (bt[None, :] == lax.broadcasted_iota(jnp.int32, (G, BLK), 0)
        ).astype(jnp.float32)
  pool_acc[...] += jnp.dot(oh, h, precision=_HI)
  gcnt_acc[...] += jnp.sum(oh, axis=1, keepdims=True)

  @pl.when(i == NB - 1)
  def _():
    pool_ref[...] = pool_acc[...] / jnp.maximum(gcnt_acc[...], 1.0)


def _tc2_body(aggp, cnt0, cnt1, h, wrel, wroot, b,
              h2_ref, stats_ref, stat_acc):
  i = pl.program_id(0)

  @pl.when(i == 0)
  def _():
    stat_acc[...] = jnp.zeros_like(stat_acc)

  cnt = cnt0[0, 0, :] + cnt1[0, 0, :]
  inv = 1.0 / jnp.maximum(cnt, 1.0)
  a = aggp[...]
  agg = (a[0] + a[1]) * inv[:, None]
  h2 = (jnp.dot(agg, wrel[...], precision=_HI)
        + jnp.dot(h[...], wroot[...], precision=_HI) + b[...])
  h2_ref[...] = h2
  stat_acc[0:1, :] += jnp.sum(h2, axis=0, keepdims=True)
  stat_acc[1:2, :] += jnp.sum(h2 * h2, axis=0, keepdims=True)

  @pl.when(i == NB - 1)
  def _():
    stats_ref[...] = stat_acc[...]


def _tc3_body(h2, stats, gamma, beta, pool1, batch, wl1a, wl1b, bl1, wl2, bl2,
              out_ref, pool_acc, gcnt_acc):
  i = pl.program_id(0)

  @pl.when(i == 0)
  def _():
    pool_acc[...] = jnp.zeros_like(pool_acc)
    gcnt_acc[...] = jnp.zeros_like(gcnt_acc)

  mu = stats[0:1, :] * (1.0 / N)
  ex2 = stats[1:2, :] * (1.0 / N)
  var = ex2 - mu * mu
  rstd = lax.rsqrt(var + 1e-5)
  h2n = (h2[...] - mu) * (rstd * gamma[...]) + beta[...]
  h2n = jnp.maximum(h2n, 0.0)

  bt = batch[0, 0, :]
  oh = (bt[None, :] == lax.broadcasted_iota(jnp.int32, (G, BLK), 0)
        ).astype(jnp.float32)
  pool_acc[...] += jnp.dot(oh, h2n, precision=_HI)
  gcnt_acc[...] += jnp.sum(oh, axis=1, keepdims=True)

  @pl.when(i == NB - 1)
  def _():
    pool2 = pool_acc[...] / jnp.maximum(gcnt_acc[...], 1.0)
    z = (jnp.dot(pool1[...], wl1a[...], precision=_HI)
         + jnp.dot(pool2, wl1b[...], precision=_HI) + bl1[...])
    z = jnp.maximum(z, 0.0)
    out_ref[...] = jnp.dot(z, wl2[...], precision=_HI) + bl2[...]


def _row_spec():
  return pl.BlockSpec((BLK, D), lambda i: (i, 0))


def _full(shape):
  return pl.BlockSpec(shape, lambda i: tuple(0 for _ in shape))


def _vec_spec():
  return pl.BlockSpec((1, 1, BLK), lambda i: (i, 0, 0))


def kernel(x, edge_index, batch, W_rel1, W_root1, b1, W_rel2, W_root2, b2,
           gamma, beta, W_lin1, b_lin1, W_lin2, b_lin2):
  pad = EPT - E // 32
  src2 = jnp.concatenate(
      [edge_index[0].reshape(32, E // 32),
       jnp.zeros((32, pad), jnp.int32)], axis=1)
  dst2 = jnp.concatenate(
      [edge_index[1].reshape(32, E // 32),
       jnp.full((32, pad), DEAD, jnp.int32)], axis=1)
  idx = (src2 | (dst2 << 16)).reshape(32, TPC, CH)
  zeros = jnp.zeros((RPT, D), jnp.float32)
  ones = jnp.ones((CH,), jnp.float32)

  aggp1, cntp = _sc_spmm_cnt(x, idx, zeros, ones)

  cnt0 = cntp[0, :N].reshape(NB, 1, BLK)
  cnt1 = cntp[1, :N].reshape(NB, 1, BLK)
  batch3 = batch.reshape(NB, 1, BLK)

  h, pool1 = pl.pallas_call(
      _tc1_body,
      grid=(NB,),
      in_specs=[
          pl.BlockSpec((2, BLK, D), lambda i: (0, i, 0)),
          _vec_spec(), _vec_spec(),
          _row_spec(),
          _vec_spec(),
          _full((D, D)), _full((D, D)), _full((1, D)),
      ],
      out_specs=[_row_spec(), _full((G, D))],
      out_shape=[jax.ShapeDtypeStruct((N, D), jnp.float32),
                 jax.ShapeDtypeStruct((G, D), jnp.float32)],
      scratch_shapes=[pltpu.VMEM((G, D), jnp.float32),
                      pltpu.VMEM((G, 1), jnp.float32)],
  )(aggp1, cnt0, cnt1, x, batch3, W_rel1, W_root1, b1.reshape(1, D))

  aggp2 = _sc_spmm(h, idx, zeros, ones)
  if isinstance(aggp2, (list, tuple)):
    aggp2 = aggp2[0]

  h2, stats = pl.pallas_call(
      _tc2_body,
      grid=(NB,),
      in_specs=[
          pl.BlockSpec((2, BLK, D), lambda i: (0, i, 0)),
          _vec_spec(), _vec_spec(),
          _row_spec(),
          _full((D, D)), _full((D, D)), _full((1, D)),
      ],
      out_specs=[_row_spec(), _full((8, D))],
      out_shape=[jax.ShapeDtypeStruct((N, D), jnp.float32),
                 jax.ShapeDtypeStruct((8, D), jnp.float32)],
      scratch_shapes=[pltpu.VMEM((8, D), jnp.float32)],
  )(aggp2, cnt0, cnt1, h, W_rel2, W_root2, b2.reshape(1, D))

  out = pl.pallas_call(
      _tc3_body,
      grid=(NB,),
      in_specs=[
          _row_spec(),
          _full((8, D)), _full((1, D)), _full((1, D)),
          _full((G, D)),
          _vec_spec(),
          _full((D, D)), _full((D, D)), _full((1, D)),
          _full((D, D)), _full((1, D)),
      ],
      out_specs=_full((G, D)),
      out_shape=jax.ShapeDtypeStruct((G, D), jnp.float32),
      scratch_shapes=[pltpu.VMEM((G, D), jnp.float32),
                      pltpu.VMEM((G, 1), jnp.float32)],
  )(h2, stats, gamma.reshape(1, D), beta.reshape(1, D), pool1, batch3,
    W_lin1[:D], W_lin1[D:], b_lin1.reshape(1, D), W_lin2,
    b_lin2.reshape(1, D))

  return out


# speedup vs baseline: 1.4864x; 1.4864x over previous
import functools

import jax
import jax.numpy as jnp
from jax import lax
from jax.experimental import pallas as pl
from jax.experimental.pallas import tpu as pltpu
from jax.experimental.pallas import tpu_sc as plsc

N = 10000
E = 320000
D = 128
G = 64

CH = 96
TPC = 105
EPT = TPC * CH
DEAD = 10016
RPT = 640
NPAD = 10240

BLK = 1000
NB = N // BLK

_HI = jax.lax.Precision.HIGHEST


NCPAD = 10240


def _make_sc_spmm(with_cnt):
  out_type = [jax.ShapeDtypeStruct((2, NPAD, D), jnp.float32)]
  scratch = [
      pltpu.VMEM((TPC, CH), jnp.int32),
      [pltpu.VMEM((CH,), jnp.int32) for _ in range(2)],
      [pltpu.VMEM((CH,), jnp.int32) for _ in range(2)],
      [pltpu.VMEM((CH, D), jnp.float32) for _ in range(2)],
      pltpu.VMEM_SHARED((NPAD, D), jnp.float32),
      [pltpu.SemaphoreType.DMA for _ in range(2)],
  ]
  if with_cnt:
    out_type.append(jax.ShapeDtypeStruct((2, NCPAD), jnp.float32))
    scratch += [
        pltpu.VMEM((CH,), jnp.float32),
        pltpu.VMEM_SHARED((NCPAD,), jnp.float32),
    ]

  mesh = plsc.VectorSubcoreMesh(core_axis_name="c", subcore_axis_name="s")

  def body(x_hbm, src_hbm, zeros_hbm, ones_hbm, *rest):
    if with_cnt:
      (acc_out, cnt_out, pk_v, ss, dd, rows, acc_sh, gsem,
       ones_v, cnt_sh) = rest
    else:
      (acc_out, pk_v, ss, dd, rows, acc_sh, gsem) = rest
    c = lax.axis_index("c")
    s = lax.axis_index("s")
    wid = c * 16 + s

    pltpu.sync_copy(zeros_hbm, acc_sh.at[pl.ds(s * RPT, RPT)])
    if with_cnt:
      pltpu.sync_copy(ones_hbm, ones_v)

      def zloop(j, _):
        idx = j * 16 + s
        pltpu.sync_copy(zeros_hbm.at[0], cnt_sh.at[pl.ds(idx * D, D)])
        return 0
      lax.fori_loop(0, NCPAD // (16 * D), zloop, 0)

    pltpu.sync_copy(src_hbm.at[wid], pk_v)
    plsc.subcore_barrier()

    def _unpack(j, k):
      for q in range(CH // 16):
        pk = pk_v[j, pl.ds(q * 16, 16)]
        ss[k][pl.ds(q * 16, 16)] = pk & 0xFFFF
        dd[k][pl.ds(q * 16, 16)] = lax.shift_right_logical(pk, 16)

    def _issue_gather(k):
      pltpu.async_copy(x_hbm.at[ss[k]], rows[k], gsem[k])

    def _wait_gather(k):
      pltpu.make_async_copy(x_hbm.at[ss[k]], rows[k], gsem[k]).wait()

    def _scatter(k):
      pltpu.sync_copy(rows[k], acc_sh.at[dd[k]], add=True)
      if with_cnt:
        pltpu.sync_copy(ones_v, cnt_sh.at[dd[k]], add=True)

    _unpack(0, 0)
    _issue_gather(0)

    def chunk2(t, _):
      ja = 2 * t
      _unpack(ja + 1, 1)
      _issue_gather(1)
      _wait_gather(0)
      _scatter(0)
      _unpack(ja + 2, 0)
      _issue_gather(0)
      _wait_gather(1)
      _scatter(1)
      return 0
    lax.fori_loop(0, (TPC - 1) // 2, chunk2, 0)

    _wait_gather(0)
    _scatter(0)

    plsc.subcore_barrier()
    pltpu.sync_copy(acc_sh.at[pl.ds(s * RPT, RPT)],
                    acc_out.at[c, pl.ds(s * RPT, RPT)])
    if with_cnt:
      @pl.when(s == 0)
      def _():
        pltpu.sync_copy(cnt_sh, cnt_out.at[c])

  return functools.partial(pl.kernel, out_type=out_type, mesh=mesh,
                           scratch_types=scratch)(body)


_sc_spmm_cnt = _make_sc_spmm(True)
_sc_spmm = _make_sc_spmm(False)


def _tc1_body(aggp, cnt0, cnt1, x, batch, wrel, wroot, b,
              h_ref, pool_ref, pool_acc, gcnt_acc):
  i = pl.program_id(0)

  @pl.when(i == 0)
  def _():
    pool_acc[...] = jnp.zeros_like(pool_acc)
    gcnt_acc[...] = jnp.zeros_like(gcnt_acc)

  cnt = cnt0[0, 0, :] + cnt1[0, 0, :]
  inv = 1.0 / jnp.maximum(cnt, 1.0)
  a = aggp[...]
  agg = (a[0] + a[1]) * inv[:, None]
  h = (jnp.dot(agg, wrel[...], precision=_HI)
       + jnp.dot(x[...], wroot[...], precision=_HI) + b[...])
  h = jnp.maximum(h, 0.0)
  h_ref[...] = h

  bt = batch[0, 0, :]
  oh = (bt[None, :] == lax.broadcasted_iota(jnp.int32, (G, BLK), 0)
        ).astype(jnp.float32)
  pool_acc[...] += jnp.dot(oh, h, precision=_HI)
  gcnt_acc[...] += jnp.sum(oh, axis=1, keepdims=True)

  @pl.when(i == NB - 1)
  def _():
    pool_ref[...] = pool_acc[...] / jnp.maximum(gcnt_acc[...], 1.0)


def _tc2_body(aggp, cnt0, cnt1, h, wrel, wroot, b,
              h2_ref, stats_ref, stat_acc):
  i = pl.program_id(0)

  @pl.when(i == 0)
  def _():
    stat_acc[...] = jnp.zeros_like(stat_acc)

  cnt = cnt0[0, 0, :] + cnt1[0, 0, :]
  inv = 1.0 / jnp.maximum(cnt, 1.0)
  a = aggp[...]
  agg = (a[0] + a[1]) * inv[:, None]
  h2 = (jnp.dot(agg, wrel[...], precision=_HI)
        + jnp.dot(h[...], wroot[...], precision=_HI) + b[...])
  h2_ref[...] = h2
  stat_acc[0:1, :] += jnp.sum(h2, axis=0, keepdims=True)
  stat_acc[1:2, :] += jnp.sum(h2 * h2, axis=0, keepdims=True)

  @pl.when(i == NB - 1)
  def _():
    stats_ref[...] = stat_acc[...]


def _tc3_body(h2, stats, gamma, beta, pool1, batch, wl1a, wl1b, bl1, wl2, bl2,
              out_ref, pool_acc, gcnt_acc):
  i = pl.program_id(0)

  @pl.when(i == 0)
  def _():
    pool_acc[...] = jnp.zeros_like(pool_acc)
    gcnt_acc[...] = jnp.zeros_like(gcnt_acc)

  mu = stats[0:1, :] * (1.0 / N)
  ex2 = stats[1:2, :] * (1.0 / N)
  var = ex2 - mu * mu
  rstd = lax.rsqrt(var + 1e-5)
  h2n = (h2[...] - mu) * (rstd * gamma[...]) + beta[...]
  h2n = jnp.maximum(h2n, 0.0)

  bt = batch[0, 0, :]
  oh = (bt[None, :] == lax.broadcasted_iota(jnp.int32, (G, BLK), 0)
        ).astype(jnp.float32)
  pool_acc[...] += jnp.dot(oh, h2n, precision=_HI)
  gcnt_acc[...] += jnp.sum(oh, axis=1, keepdims=True)

  @pl.when(i == NB - 1)
  def _():
    pool2 = pool_acc[...] / jnp.maximum(gcnt_acc[...], 1.0)
    z = (jnp.dot(pool1[...], wl1a[...], precision=_HI)
         + jnp.dot(pool2, wl1b[...], precision=_HI) + bl1[...])
    z = jnp.maximum(z, 0.0)
    out_ref[...] = jnp.dot(z, wl2[...], precision=_HI) + bl2[...]


def _row_spec():
  return pl.BlockSpec((BLK, D), lambda i: (i, 0))


def _full(shape):
  return pl.BlockSpec(shape, lambda i: tuple(0 for _ in shape))


def _vec_spec():
  return pl.BlockSpec((1, 1, BLK), lambda i: (i, 0, 0))


def kernel(x, edge_index, batch, W_rel1, W_root1, b1, W_rel2, W_root2, b2,
           gamma, beta, W_lin1, b_lin1, W_lin2, b_lin2):
  pad = EPT - E // 32
  src2 = jnp.concatenate(
      [edge_index[0].reshape(32, E // 32),
       jnp.zeros((32, pad), jnp.int32)], axis=1)
  dst2 = jnp.concatenate(
      [edge_index[1].reshape(32, E // 32),
       jnp.full((32, pad), DEAD, jnp.int32)], axis=1)
  idx = (src2 | (dst2 << 16)).reshape(32, TPC, CH)
  zeros = jnp.zeros((RPT, D), jnp.float32)
  ones = jnp.ones((CH,), jnp.float32)

  aggp1, cntp = _sc_spmm_cnt(x, idx, zeros, ones)

  cnt0 = cntp[0, :N].reshape(NB, 1, BLK)
  cnt1 = cntp[1, :N].reshape(NB, 1, BLK)
  batch3 = batch.reshape(NB, 1, BLK)

  h, pool1 = pl.pallas_call(
      _tc1_body,
      grid=(NB,),
      in_specs=[
          pl.BlockSpec((2, BLK, D), lambda i: (0, i, 0)),
          _vec_spec(), _vec_spec(),
          _row_spec(),
          _vec_spec(),
          _full((D, D)), _full((D, D)), _full((1, D)),
      ],
      out_specs=[_row_spec(), _full((G, D))],
      out_shape=[jax.ShapeDtypeStruct((N, D), jnp.float32),
                 jax.ShapeDtypeStruct((G, D), jnp.float32)],
      scratch_shapes=[pltpu.VMEM((G, D), jnp.float32),
                      pltpu.VMEM((G, 1), jnp.float32)],
  )(aggp1, cnt0, cnt1, x, batch3, W_rel1, W_root1, b1.reshape(1, D))

  aggp2 = _sc_spmm(h, idx, zeros, ones)
  if isinstance(aggp2, (list, tuple)):
    aggp2 = aggp2[0]

  h2, stats = pl.pallas_call(
      _tc2_body,
      grid=(NB,),
      in_specs=[
          pl.BlockSpec((2, BLK, D), lambda i: (0, i, 0)),
          _vec_spec(), _vec_spec(),
          _row_spec(),
          _full((D, D)), _full((D, D)), _full((1, D)),
      ],
      out_specs=[_row_spec(), _full((8, D))],
      out_shape=[jax.ShapeDtypeStruct((N, D), jnp.float32),
                 jax.ShapeDtypeStruct((8, D), jnp.float32)],
      scratch_shapes=[pltpu.VMEM((8, D), jnp.float32)],
  )(aggp2, cnt0, cnt1, h, W_rel2, W_root2, b2.reshape(1, D))

  out = pl.pallas_call(
      _tc3_body,
      grid=(NB,),
      in_specs=[
          _row_spec(),
          _full((8, D)), _full((1, D)), _full((1, D)),
          _full((G, D)),
          _vec_spec(),
          _full((D, D)), _full((D, D)), _full((1, D)),
          _full((D, D)), _full((1, D)),
      ],
      out_specs=_full((G, D)),
      out_shape=jax.ShapeDtypeStruct((G, D), jnp.float32),
      scratch_shapes=[pltpu.VMEM((G, D), jnp.float32),
                      pltpu.VMEM((G, 1), jnp.float32)],
  )(h2, stats, gamma.reshape(1, D), beta.reshape(1, D), pool1, batch3,
    W_lin1[:D], W_lin1[D:], b_lin1.reshape(1, D), W_lin2,
    b_lin2.reshape(1, D))

  return out


# speedup vs baseline: 2.2729x; 1.5292x over previous
import functools

import jax
import jax.numpy as jnp
from jax import lax
from jax.experimental import pallas as pl
from jax.experimental.pallas import tpu as pltpu
from jax.experimental.pallas import tpu_sc as plsc

N = 10000
E = 320000
D = 128
G = 64

CH = 80
TPC = 125
EPT = TPC * CH
DEAD = 10016
RPT = 640
NPAD = 10240

BLK = 1000
NB = N // BLK

_HI = jax.lax.Precision.HIGHEST


NCPAD = 10240


def _make_sc_spmm(with_cnt):
  out_type = [jax.ShapeDtypeStruct((2, NPAD, D), jnp.float32)]
  scratch = [
      pltpu.VMEM((TPC, CH), jnp.int32),
      [pltpu.VMEM((CH,), jnp.int32) for _ in range(2)],
      [pltpu.VMEM((CH,), jnp.int32) for _ in range(2)],
      [pltpu.VMEM((CH, D), jnp.float32) for _ in range(2)],
      pltpu.VMEM_SHARED((NPAD, D), jnp.float32),
      [pltpu.SemaphoreType.DMA for _ in range(2)],
  ]
  if with_cnt:
    out_type.append(jax.ShapeDtypeStruct((2, NCPAD), jnp.float32))
    scratch += [
        pltpu.VMEM((CH,), jnp.float32),
        pltpu.VMEM_SHARED((NCPAD,), jnp.float32),
    ]

  mesh = plsc.VectorSubcoreMesh(core_axis_name="c", subcore_axis_name="s")

  def body(x_hbm, src_hbm, zeros_hbm, ones_hbm, *rest):
    if with_cnt:
      (acc_out, cnt_out, pk_v, ss, dd, rows, acc_sh, gsem,
       ones_v, cnt_sh) = rest
    else:
      (acc_out, pk_v, ss, dd, rows, acc_sh, gsem) = rest
    c = lax.axis_index("c")
    s = lax.axis_index("s")
    wid = c * 16 + s

    pltpu.sync_copy(zeros_hbm, acc_sh.at[pl.ds(s * RPT, RPT)])
    if with_cnt:
      pltpu.sync_copy(ones_hbm, ones_v)

      def zloop(j, _):
        idx = j * 16 + s
        pltpu.sync_copy(zeros_hbm.at[0], cnt_sh.at[pl.ds(idx * D, D)])
        return 0
      lax.fori_loop(0, NCPAD // (16 * D), zloop, 0)

    pltpu.sync_copy(src_hbm.at[wid], pk_v)
    plsc.subcore_barrier()

    def _unpack(j, k):
      for q in range(CH // 16):
        pk = pk_v[j, pl.ds(q * 16, 16)]
        ss[k][pl.ds(q * 16, 16)] = pk & 0xFFFF
        dd[k][pl.ds(q * 16, 16)] = lax.shift_right_logical(pk, 16)

    def _issue_gather(k):
      pltpu.async_copy(x_hbm.at[ss[k]], rows[k], gsem[k])

    def _wait_gather(k):
      pltpu.make_async_copy(x_hbm.at[ss[k]], rows[k], gsem[k]).wait()

    def _scatter(k):
      pltpu.sync_copy(rows[k], acc_sh.at[dd[k]], add=True)
      if with_cnt:
        pltpu.sync_copy(ones_v, cnt_sh.at[dd[k]], add=True)

    _unpack(0, 0)
    _issue_gather(0)

    def chunk2(t, _):
      ja = 2 * t
      _unpack(ja + 1, 1)
      _issue_gather(1)
      _wait_gather(0)
      _scatter(0)
      _unpack(ja + 2, 0)
      _issue_gather(0)
      _wait_gather(1)
      _scatter(1)
      return 0
    lax.fori_loop(0, (TPC - 1) // 2, chunk2, 0)

    _wait_gather(0)
    _scatter(0)

    plsc.subcore_barrier()
    pltpu.sync_copy(acc_sh.at[pl.ds(s * RPT, RPT)],
                    acc_out.at[c, pl.ds(s * RPT, RPT)])
    if with_cnt:
      @pl.when(s == 0)
      def _():
        pltpu.sync_copy(cnt_sh, cnt_out.at[c])

  return functools.partial(pl.kernel, out_type=out_type, mesh=mesh,
                           scratch_types=scratch)(body)


_sc_spmm_cnt = _make_sc_spmm(True)
_sc_spmm = _make_sc_spmm(False)


def _tc1_body(aggp, cnt0, cnt1, x, batch, wrel, wroot, b,
              h_ref, pool_ref, pool_acc, gcnt_acc):
  i = pl.program_id(0)

  @pl.when(i == 0)
  def _():
    pool_acc[...] = jnp.zeros_like(pool_acc)
    gcnt_acc[...] = jnp.zeros_like(gcnt_acc)

  cnt = cnt0[0, 0, :] + cnt1[0, 0, :]
  inv = 1.0 / jnp.maximum(cnt, 1.0)
  a = aggp[...]
  agg = (a[0] + a[1]) * inv[:, None]
  h = (jnp.dot(agg, wrel[...], precision=_HI)
       + jnp.dot(x[...], wroot[...], precision=_HI) + b[...])
  h = jnp.maximum(h, 0.0)
  h_ref[...] = h

  bt = batch[0, 0, :]
  oh = (bt[None, :] == lax.broadcasted_iota(jnp.int32, (G, BLK), 0)
        ).astype(jnp.float32)
  pool_acc[...] += jnp.dot(oh, h, precision=_HI)
  gcnt_acc[...] += jnp.sum(oh, axis=1, keepdims=True)

  @pl.when(i == NB - 1)
  def _():
    pool_ref[...] = pool_acc[...] / jnp.maximum(gcnt_acc[...], 1.0)


def _tc23_body(aggp, cnt0, cnt1, h, wrel, wroot, b, gamma, beta, pool1,
               batch, wl1a, wl1b, bl1, wl2, bl2,
               out_ref, h2_s, stat_acc, pool_acc, gcnt_acc):
  p = pl.program_id(0)
  i = pl.program_id(1)

  @pl.when((p == 0) & (i == 0))
  def _():
    stat_acc[...] = jnp.zeros_like(stat_acc)
    pool_acc[...] = jnp.zeros_like(pool_acc)
    gcnt_acc[...] = jnp.zeros_like(gcnt_acc)

  @pl.when(p == 0)
  def _():
    cnt = cnt0[0, 0, :] + cnt1[0, 0, :]
    inv = 1.0 / jnp.maximum(cnt, 1.0)
    a = aggp[...]
    agg = (a[0] + a[1]) * inv[:, None]
    h2 = (jnp.dot(agg, wrel[...], precision=_HI)
          + jnp.dot(h[...], wroot[...], precision=_HI) + b[...])
    h2_s[pl.ds(i * BLK, BLK), :] = h2
    stat_acc[0:1, :] += jnp.sum(h2, axis=0, keepdims=True)
    stat_acc[1:2, :] += jnp.sum(h2 * h2, axis=0, keepdims=True)

  @pl.when(p == 1)
  def _():
    mu = stat_acc[0:1, :] * (1.0 / N)
    ex2 = stat_acc[1:2, :] * (1.0 / N)
    var = ex2 - mu * mu
    rstd = lax.rsqrt(var + 1e-5)
    h2 = h2_s[pl.ds(i * BLK, BLK), :]
    h2n = (h2 - mu) * (rstd * gamma[...]) + beta[...]
    h2n = jnp.maximum(h2n, 0.0)
    bt = batch[0, 0, :]
    oh = (bt[None, :] == lax.broadcasted_iota(jnp.int32, (G, BLK), 0)
          ).astype(jnp.float32)
    pool_acc[...] += jnp.dot(oh, h2n, precision=_HI)
    gcnt_acc[...] += jnp.sum(oh, axis=1, keepdims=True)

  @pl.when((p == 1) & (i == NB - 1))
  def _():
    pool2 = pool_acc[...] / jnp.maximum(gcnt_acc[...], 1.0)
    z = (jnp.dot(pool1[...], wl1a[...], precision=_HI)
         + jnp.dot(pool2, wl1b[...], precision=_HI) + bl1[...])
    z = jnp.maximum(z, 0.0)
    out_ref[...] = jnp.dot(z, wl2[...], precision=_HI) + bl2[...]


def _row_spec():
  return pl.BlockSpec((BLK, D), lambda i: (i, 0))


def _full(shape):
  return pl.BlockSpec(shape, lambda i: tuple(0 for _ in shape))


def _vec_spec():
  return pl.BlockSpec((1, 1, BLK), lambda i: (i, 0, 0))


def kernel(x, edge_index, batch, W_rel1, W_root1, b1, W_rel2, W_root2, b2,
           gamma, beta, W_lin1, b_lin1, W_lin2, b_lin2):
  src2 = edge_index[0].reshape(32, TPC, CH)
  dst2 = edge_index[1].reshape(32, TPC, CH)
  idx = src2 | (dst2 << 16)
  zeros = jnp.zeros((RPT, D), jnp.float32)
  ones = jnp.ones((CH,), jnp.float32)

  aggp1, cntp = _sc_spmm_cnt(x, idx, zeros, ones)

  cnt0 = cntp[0, :N].reshape(NB, 1, BLK)
  cnt1 = cntp[1, :N].reshape(NB, 1, BLK)
  batch3 = batch.reshape(NB, 1, BLK)

  h, pool1 = pl.pallas_call(
      _tc1_body,
      grid=(NB,),
      in_specs=[
          pl.BlockSpec((2, BLK, D), lambda i: (0, i, 0)),
          _vec_spec(), _vec_spec(),
          _row_spec(),
          _vec_spec(),
          _full((D, D)), _full((D, D)), _full((1, D)),
      ],
      out_specs=[_row_spec(), _full((G, D))],
      out_shape=[jax.ShapeDtypeStruct((N, D), jnp.float32),
                 jax.ShapeDtypeStruct((G, D), jnp.float32)],
      scratch_shapes=[pltpu.VMEM((G, D), jnp.float32),
                      pltpu.VMEM((G, 1), jnp.float32)],
  )(aggp1, cnt0, cnt1, x, batch3, W_rel1, W_root1, b1.reshape(1, D))

  aggp2 = _sc_spmm(h, idx, zeros, ones)
  if isinstance(aggp2, (list, tuple)):
    aggp2 = aggp2[0]

  row23 = lambda p, i: (i * (1 - p) + (NB - 1) * p, 0)
  out = pl.pallas_call(
      _tc23_body,
      grid=(2, NB),
      in_specs=[
          pl.BlockSpec((2, BLK, D),
                       lambda p, i: (0, i * (1 - p) + (NB - 1) * p, 0)),
          pl.BlockSpec((1, 1, BLK),
                       lambda p, i: (i * (1 - p) + (NB - 1) * p, 0, 0)),
          pl.BlockSpec((1, 1, BLK),
                       lambda p, i: (i * (1 - p) + (NB - 1) * p, 0, 0)),
          pl.BlockSpec((BLK, D), row23),
          pl.BlockSpec((D, D), lambda p, i: (0, 0)),
          pl.BlockSpec((D, D), lambda p, i: (0, 0)),
          pl.BlockSpec((1, D), lambda p, i: (0, 0)),
          pl.BlockSpec((1, D), lambda p, i: (0, 0)),
          pl.BlockSpec((1, D), lambda p, i: (0, 0)),
          pl.BlockSpec((G, D), lambda p, i: (0, 0)),
          pl.BlockSpec((1, 1, BLK), lambda p, i: (i * p, 0, 0)),
          pl.BlockSpec((D, D), lambda p, i: (0, 0)),
          pl.BlockSpec((D, D), lambda p, i: (0, 0)),
          pl.BlockSpec((1, D), lambda p, i: (0, 0)),
          pl.BlockSpec((D, D), lambda p, i: (0, 0)),
          pl.BlockSpec((1, D), lambda p, i: (0, 0)),
      ],
      out_specs=pl.BlockSpec((G, D), lambda p, i: (0, 0)),
      out_shape=jax.ShapeDtypeStruct((G, D), jnp.float32),
      scratch_shapes=[pltpu.VMEM((N, D), jnp.float32),
                      pltpu.VMEM((8, D), jnp.float32),
                      pltpu.VMEM((G, D), jnp.float32),
                      pltpu.VMEM((G, 1), jnp.float32)],
  )(aggp2, cnt0, cnt1, h, W_rel2, W_root2, b2.reshape(1, D),
    gamma.reshape(1, D), beta.reshape(1, D), pool1, batch3,
    W_lin1[:D], W_lin1[D:], b_lin1.reshape(1, D), W_lin2,
    b_lin2.reshape(1, D))

  return out


# speedup vs baseline: 2.2798x; 1.0030x over previous
import functools

import jax
import jax.numpy as jnp
from jax import lax
from jax.experimental import pallas as pl
from jax.experimental.pallas import tpu as pltpu
from jax.experimental.pallas import tpu_sc as plsc

N = 10000
E = 320000
D = 128
G = 64

CH = 80
TPC = 125
EPT = TPC * CH
DEAD = 10016
RPT = 640
NPAD = 10240

BLK = 1000
NB = N // BLK

_HI = jax.lax.Precision.HIGHEST


NCPAD = 10240


def _make_sc_spmm(with_cnt):
  out_type = [jax.ShapeDtypeStruct((2, NPAD, D), jnp.float32)]
  scratch = [
      pltpu.VMEM((TPC, CH), jnp.int32),
      [pltpu.VMEM((CH,), jnp.int32) for _ in range(2)],
      [pltpu.VMEM((CH,), jnp.int32) for _ in range(2)],
      [pltpu.VMEM((CH, D), jnp.float32) for _ in range(2)],
      pltpu.VMEM_SHARED((NPAD, D), jnp.float32),
      [pltpu.SemaphoreType.DMA for _ in range(2)],
  ]
  if with_cnt:
    out_type.append(jax.ShapeDtypeStruct((2, NCPAD), jnp.float32))
    scratch += [
        pltpu.VMEM((CH,), jnp.float32),
        pltpu.VMEM_SHARED((NCPAD,), jnp.float32),
    ]

  mesh = plsc.VectorSubcoreMesh(core_axis_name="c", subcore_axis_name="s")

  def body(x_hbm, src_hbm, zeros_hbm, ones_hbm, *rest):
    if with_cnt:
      (acc_out, cnt_out, pk_v, ss, dd, rows, acc_sh, gsem,
       ones_v, cnt_sh) = rest
    else:
      (acc_out, pk_v, ss, dd, rows, acc_sh, gsem) = rest
    c = lax.axis_index("c")
    s = lax.axis_index("s")
    wid = c * 16 + s

    pltpu.sync_copy(zeros_hbm, acc_sh.at[pl.ds(s * RPT, RPT)])
    if with_cnt:
      pltpu.sync_copy(ones_hbm, ones_v)

      def zloop(j, _):
        idx = j * 16 + s
        pltpu.sync_copy(zeros_hbm.at[0], cnt_sh.at[pl.ds(idx * D, D)])
        return 0
      lax.fori_loop(0, NCPAD // (16 * D), zloop, 0)

    pltpu.sync_copy(src_hbm.at[wid], pk_v)
    plsc.subcore_barrier()

    def _unpack(j, k):
      for q in range(CH // 16):
        pk = pk_v[j, pl.ds(q * 16, 16)]
        ss[k][pl.ds(q * 16, 16)] = pk & 0xFFFF
        dd[k][pl.ds(q * 16, 16)] = lax.shift_right_logical(pk, 16)

    def _issue_gather(k):
      pltpu.async_copy(x_hbm.at[ss[k]], rows[k], gsem[k])

    def _wait_gather(k):
      pltpu.make_async_copy(x_hbm.at[ss[k]], rows[k], gsem[k]).wait()

    def _scatter(k):
      pltpu.sync_copy(rows[k], acc_sh.at[dd[k]], add=True)
      if with_cnt:
        pltpu.sync_copy(ones_v, cnt_sh.at[dd[k]], add=True)

    _unpack(0, 0)
    _issue_gather(0)

    def chunk2(t, _):
      ja = 2 * t
      _unpack(ja + 1, 1)
      _issue_gather(1)
      _wait_gather(0)
      _scatter(0)
      _unpack(ja + 2, 0)
      _issue_gather(0)
      _wait_gather(1)
      _scatter(1)
      return 0
    lax.fori_loop(0, (TPC - 1) // 2, chunk2, 0)

    _wait_gather(0)
    _scatter(0)

    plsc.subcore_barrier()
    pltpu.sync_copy(acc_sh.at[pl.ds(s * RPT, RPT)],
                    acc_out.at[c, pl.ds(s * RPT, RPT)])
    if with_cnt:
      @pl.when(s == 0)
      def _():
        pltpu.sync_copy(cnt_sh, cnt_out.at[c])

  return functools.partial(pl.kernel, out_type=out_type, mesh=mesh,
                           scratch_types=scratch)(body)


_sc_spmm_cnt = _make_sc_spmm(True)
_sc_spmm = _make_sc_spmm(False)


def _xr_body(x, wroot, b, xr_ref):
  xr_ref[...] = jnp.dot(x[...], wroot[...], precision=_HI) + b[...]


def _tc1_body(aggp, cnt0, cnt1, xr, batch, wrel,
              h_ref, pool_ref, pool_acc, gcnt_acc):
  i = pl.program_id(0)

  @pl.when(i == 0)
  def _():
    pool_acc[...] = jnp.zeros_like(pool_acc)
    gcnt_acc[...] = jnp.zeros_like(gcnt_acc)

  cnt = cnt0[0, 0, :] + cnt1[0, 0, :]
  inv = 1.0 / jnp.maximum(cnt, 1.0)
  a = aggp[...]
  agg = (a[0] + a[1]) * inv[:, None]
  h = jnp.dot(agg, wrel[...], precision=_HI) + xr[...]
  h = jnp.maximum(h, 0.0)
  h_ref[...] = h

  bt = batch[0, 0, :]
  oh = (bt[None, :] == lax.broadcasted_iota(jnp.int32, (G, BLK), 0)
        ).astype(jnp.float32)
  pool_acc[...] += jnp.dot(oh, h, precision=_HI)
  gcnt_acc[...] += jnp.sum(oh, axis=1, keepdims=True)

  @pl.when(i == NB - 1)
  def _():
    pool_ref[...] = pool_acc[...] / jnp.maximum(gcnt_acc[...], 1.0)


def _tc23_body(aggp, cnt0, cnt1, h, wrel, wroot, b, gamma, beta, pool1,
               batch, wl1a, wl1b, bl1, wl2, bl2,
               out_ref, h2_s, stat_acc, pool_acc, gcnt_acc):
  p = pl.program_id(0)
  i = pl.program_id(1)

  @pl.when((p == 0) & (i == 0))
  def _():
    stat_acc[...] = jnp.zeros_like(stat_acc)
    pool_acc[...] = jnp.zeros_like(pool_acc)
    gcnt_acc[...] = jnp.zeros_like(gcnt_acc)

  @pl.when(p == 0)
  def _():
    cnt = cnt0[0, 0, :] + cnt1[0, 0, :]
    inv = 1.0 / jnp.maximum(cnt, 1.0)
    a = aggp[...]
    agg = (a[0] + a[1]) * inv[:, None]
    h2 = (jnp.dot(agg, wrel[...], precision=_HI)
          + jnp.dot(h[...], wroot[...], precision=_HI) + b[...])
    h2_s[pl.ds(i * BLK, BLK), :] = h2
    stat_acc[0:1, :] += jnp.sum(h2, axis=0, keepdims=True)
    stat_acc[1:2, :] += jnp.sum(h2 * h2, axis=0, keepdims=True)

  @pl.when(p == 1)
  def _():
    mu = stat_acc[0:1, :] * (1.0 / N)
    ex2 = stat_acc[1:2, :] * (1.0 / N)
    var = ex2 - mu * mu
    rstd = lax.rsqrt(var + 1e-5)
    h2 = h2_s[pl.ds(i * BLK, BLK), :]
    h2n = (h2 - mu) * (rstd * gamma[...]) + beta[...]
    h2n = jnp.maximum(h2n, 0.0)
    bt = batch[0, 0, :]
    oh = (bt[None, :] == lax.broadcasted_iota(jnp.int32, (G, BLK), 0)
          ).astype(jnp.float32)
    pool_acc[...] += jnp.dot(oh, h2n, precision=_HI)
    gcnt_acc[...] += jnp.sum(oh, axis=1, keepdims=True)

  @pl.when((p == 1) & (i == NB - 1))
  def _():
    pool2 = pool_acc[...] / jnp.maximum(gcnt_acc[...], 1.0)
    z = (jnp.dot(pool1[...], wl1a[...], precision=_HI)
         + jnp.dot(pool2, wl1b[...], precision=_HI) + bl1[...])
    z = jnp.maximum(z, 0.0)
    out_ref[...] = jnp.dot(z, wl2[...], precision=_HI) + bl2[...]


def _row_spec():
  return pl.BlockSpec((BLK, D), lambda i: (i, 0))


def _full(shape):
  return pl.BlockSpec(shape, lambda i: tuple(0 for _ in shape))


def _vec_spec():
  return pl.BlockSpec((1, 1, BLK), lambda i: (i, 0, 0))


def kernel(x, edge_index, batch, W_rel1, W_root1, b1, W_rel2, W_root2, b2,
           gamma, beta, W_lin1, b_lin1, W_lin2, b_lin2):
  src2 = edge_index[0].reshape(32, TPC, CH)
  dst2 = edge_index[1].reshape(32, TPC, CH)
  idx = src2 | (dst2 << 16)
  zeros = jnp.zeros((RPT, D), jnp.float32)
  ones = jnp.ones((CH,), jnp.float32)

  xr = pl.pallas_call(
      _xr_body,
      grid=(NB,),
      in_specs=[_row_spec(), _full((D, D)), _full((1, D))],
      out_specs=_row_spec(),
      out_shape=jax.ShapeDtypeStruct((N, D), jnp.float32),
  )(x, W_root1, b1.reshape(1, D))

  aggp1, cntp = _sc_spmm_cnt(x, idx, zeros, ones)

  cnt0 = cntp[0, :N].reshape(NB, 1, BLK)
  cnt1 = cntp[1, :N].reshape(NB, 1, BLK)
  batch3 = batch.reshape(NB, 1, BLK)

  h, pool1 = pl.pallas_call(
      _tc1_body,
      grid=(NB,),
      in_specs=[
          pl.BlockSpec((2, BLK, D), lambda i: (0, i, 0)),
          _vec_spec(), _vec_spec(),
          _row_spec(),
          _vec_spec(),
          _full((D, D)),
      ],
      out_specs=[_row_spec(), _full((G, D))],
      out_shape=[jax.ShapeDtypeStruct((N, D), jnp.float32),
                 jax.ShapeDtypeStruct((G, D), jnp.float32)],
      scratch_shapes=[pltpu.VMEM((G, D), jnp.float32),
                      pltpu.VMEM((G, 1), jnp.float32)],
  )(aggp1, cnt0, cnt1, xr, batch3, W_rel1)

  aggp2 = _sc_spmm(h, idx, zeros, ones)
  if isinstance(aggp2, (list, tuple)):
    aggp2 = aggp2[0]

  row23 = lambda p, i: (i * (1 - p) + (NB - 1) * p, 0)
  out = pl.pallas_call(
      _tc23_body,
      grid=(2, NB),
      in_specs=[
          pl.BlockSpec((2, BLK, D),
                       lambda p, i: (0, i * (1 - p) + (NB - 1) * p, 0)),
          pl.BlockSpec((1, 1, BLK),
                       lambda p, i: (i * (1 - p) + (NB - 1) * p, 0, 0)),
          pl.BlockSpec((1, 1, BLK),
                       lambda p, i: (i * (1 - p) + (NB - 1) * p, 0, 0)),
          pl.BlockSpec((BLK, D), row23),
          pl.BlockSpec((D, D), lambda p, i: (0, 0)),
          pl.BlockSpec((D, D), lambda p, i: (0, 0)),
          pl.BlockSpec((1, D), lambda p, i: (0, 0)),
          pl.BlockSpec((1, D), lambda p, i: (0, 0)),
          pl.BlockSpec((1, D), lambda p, i: (0, 0)),
          pl.BlockSpec((G, D), lambda p, i: (0, 0)),
          pl.BlockSpec((1, 1, BLK), lambda p, i: (i * p, 0, 0)),
          pl.BlockSpec((D, D), lambda p, i: (0, 0)),
          pl.BlockSpec((D, D), lambda p, i: (0, 0)),
          pl.BlockSpec((1, D), lambda p, i: (0, 0)),
          pl.BlockSpec((D, D), lambda p, i: (0, 0)),
          pl.BlockSpec((1, D), lambda p, i: (0, 0)),
      ],
      out_specs=pl.BlockSpec((G, D), lambda p, i: (0, 0)),
      out_shape=jax.ShapeDtypeStruct((G, D), jnp.float32),
      scratch_shapes=[pltpu.VMEM((N, D), jnp.float32),
                      pltpu.VMEM((8, D), jnp.float32),
                      pltpu.VMEM((G, D), jnp.float32),
                      pltpu.VMEM((G, 1), jnp.float32)],
  )(aggp2, cnt0, cnt1, h, W_rel2, W_root2, b2.reshape(1, D),
    gamma.reshape(1, D), beta.reshape(1, D), pool1, batch3,
    W_lin1[:D], W_lin1[D:], b_lin1.reshape(1, D), W_lin2,
    b_lin2.reshape(1, D))

  return out


# speedup vs baseline: 2.3051x; 1.0111x over previous
import functools

import jax
import jax.numpy as jnp
from jax import lax
from jax.experimental import pallas as pl
from jax.experimental.pallas import tpu as pltpu
from jax.experimental.pallas import tpu_sc as plsc

N = 10000
E = 320000
D = 128
G = 64

CH = 80
TPC = 125
EPT = TPC * CH
DEAD = 10016
RPT = 640
NPAD = 10240

BLK = 1000
NB = N // BLK

_HI = jax.lax.Precision.HIGHEST


NCPAD = 10240


def _make_sc_spmm(with_cnt):
  out_type = [jax.ShapeDtypeStruct((2, NPAD, D), jnp.float32)]
  scratch = [
      pltpu.VMEM((TPC, CH), jnp.int32),
      [pltpu.VMEM((CH,), jnp.int32) for _ in range(2)],
      [pltpu.VMEM((CH,), jnp.int32) for _ in range(2)],
      [pltpu.VMEM((CH, D), jnp.float32) for _ in range(2)],
      pltpu.VMEM_SHARED((NPAD, D), jnp.float32),
      [pltpu.SemaphoreType.DMA for _ in range(2)],
  ]
  if with_cnt:
    out_type.append(jax.ShapeDtypeStruct((2, NCPAD), jnp.float32))
    scratch += [
        pltpu.VMEM((CH,), jnp.float32),
        pltpu.VMEM_SHARED((NCPAD,), jnp.float32),
    ]

  mesh = plsc.VectorSubcoreMesh(core_axis_name="c", subcore_axis_name="s")

  def body(x_hbm, src_hbm, zeros_hbm, ones_hbm, *rest):
    if with_cnt:
      (acc_out, cnt_out, pk_v, ss, dd, rows, acc_sh, gsem,
       ones_v, cnt_sh) = rest
    else:
      (acc_out, pk_v, ss, dd, rows, acc_sh, gsem) = rest
    c = lax.axis_index("c")
    s = lax.axis_index("s")
    wid = c * 16 + s

    pltpu.sync_copy(zeros_hbm, acc_sh.at[pl.ds(s * RPT, RPT)])
    if with_cnt:
      pltpu.sync_copy(ones_hbm, ones_v)

      def zloop(j, _):
        idx = j * 16 + s
        pltpu.sync_copy(zeros_hbm.at[0], cnt_sh.at[pl.ds(idx * D, D)])
        return 0
      lax.fori_loop(0, NCPAD // (16 * D), zloop, 0)

    pltpu.sync_copy(src_hbm.at[wid], pk_v)
    plsc.subcore_barrier()

    def _unpack(j, k):
      for q in range(CH // 16):
        pk = pk_v[j, pl.ds(q * 16, 16)]
        ss[k][pl.ds(q * 16, 16)] = pk & 0xFFFF
        dd[k][pl.ds(q * 16, 16)] = lax.shift_right_logical(pk, 16)

    def _issue_gather(k):
      pltpu.async_copy(x_hbm.at[ss[k]], rows[k], gsem[k])

    def _wait_gather(k):
      pltpu.make_async_copy(x_hbm.at[ss[k]], rows[k], gsem[k]).wait()

    def _scatter(k):
      if with_cnt:
        cdesc = pltpu.async_copy(ones_v, cnt_sh.at[dd[k]], gsem[k], add=True)
        pltpu.sync_copy(rows[k], acc_sh.at[dd[k]], add=True)
        cdesc.wait()
      else:
        pltpu.sync_copy(rows[k], acc_sh.at[dd[k]], add=True)

    _unpack(0, 0)
    _issue_gather(0)

    def chunk2(t, _):
      ja = 2 * t
      _unpack(ja + 1, 1)
      _issue_gather(1)
      _wait_gather(0)
      _scatter(0)
      _unpack(ja + 2, 0)
      _issue_gather(0)
      _wait_gather(1)
      _scatter(1)
      return 0
    lax.fori_loop(0, (TPC - 1) // 2, chunk2, 0)

    _wait_gather(0)
    _scatter(0)

    plsc.subcore_barrier()
    pltpu.sync_copy(acc_sh.at[pl.ds(s * RPT, RPT)],
                    acc_out.at[c, pl.ds(s * RPT, RPT)])
    if with_cnt:
      @pl.when(s == 0)
      def _():
        pltpu.sync_copy(cnt_sh, cnt_out.at[c])

  return functools.partial(pl.kernel, out_type=out_type, mesh=mesh,
                           scratch_types=scratch)(body)


_sc_spmm_cnt = _make_sc_spmm(True)
_sc_spmm = _make_sc_spmm(False)


def _xr_body(x, wroot, b, xr_ref):
  xr_ref[...] = jnp.dot(x[...], wroot[...], precision=_HI) + b[...]


def _tc1_body(aggp, cnt0, cnt1, xr, wrel, h_ref):
  cnt = cnt0[0, 0, :] + cnt1[0, 0, :]
  inv = 1.0 / jnp.maximum(cnt, 1.0)
  a = aggp[...]
  agg = (a[0] + a[1]) * inv[:, None]
  h = jnp.dot(agg, wrel[...], precision=_HI) + xr[...]
  h_ref[...] = jnp.maximum(h, 0.0)


def _tc23_body(aggp, cnt0, cnt1, h, wrel, wroot, b, gamma, beta,
               batch, wl1a, wl1b, bl1, wl2, bl2,
               out_ref, h2_s, stat_acc, pool1_acc, pool_acc, gcnt_acc):
  p = pl.program_id(0)
  i = pl.program_id(1)

  @pl.when((p == 0) & (i == 0))
  def _():
    stat_acc[...] = jnp.zeros_like(stat_acc)
    pool1_acc[...] = jnp.zeros_like(pool1_acc)
    pool_acc[...] = jnp.zeros_like(pool_acc)
    gcnt_acc[...] = jnp.zeros_like(gcnt_acc)

  bt = batch[0, 0, :]
  oh = (bt[None, :] == lax.broadcasted_iota(jnp.int32, (G, BLK), 0)
        ).astype(jnp.float32)

  @pl.when(p == 0)
  def _():
    cnt = cnt0[0, 0, :] + cnt1[0, 0, :]
    inv = 1.0 / jnp.maximum(cnt, 1.0)
    a = aggp[...]
    agg = (a[0] + a[1]) * inv[:, None]
    hb = h[...]
    h2 = (jnp.dot(agg, wrel[...], precision=_HI)
          + jnp.dot(hb, wroot[...], precision=_HI) + b[...])
    h2_s[pl.ds(i * BLK, BLK), :] = h2
    stat_acc[0:1, :] += jnp.sum(h2, axis=0, keepdims=True)
    stat_acc[1:2, :] += jnp.sum(h2 * h2, axis=0, keepdims=True)
    pool1_acc[...] += jnp.dot(oh, hb, precision=_HI)
    gcnt_acc[...] += jnp.sum(oh, axis=1, keepdims=True)

  @pl.when(p == 1)
  def _():
    mu = stat_acc[0:1, :] * (1.0 / N)
    ex2 = stat_acc[1:2, :] * (1.0 / N)
    var = ex2 - mu * mu
    rstd = lax.rsqrt(var + 1e-5)
    h2 = h2_s[pl.ds(i * BLK, BLK), :]
    h2n = (h2 - mu) * (rstd * gamma[...]) + beta[...]
    h2n = jnp.maximum(h2n, 0.0)
    pool_acc[...] += jnp.dot(oh, h2n, precision=_HI)

  @pl.when((p == 1) & (i == NB - 1))
  def _():
    gc = jnp.maximum(gcnt_acc[...], 1.0)
    pool1 = pool1_acc[...] / gc
    pool2 = pool_acc[...] / gc
    z = (jnp.dot(pool1, wl1a[...], precision=_HI)
         + jnp.dot(pool2, wl1b[...], precision=_HI) + bl1[...])
    z = jnp.maximum(z, 0.0)
    out_ref[...] = jnp.dot(z, wl2[...], precision=_HI) + bl2[...]


def _row_spec():
  return pl.BlockSpec((BLK, D), lambda i: (i, 0))


def _full(shape):
  return pl.BlockSpec(shape, lambda i: tuple(0 for _ in shape))


def _vec_spec():
  return pl.BlockSpec((1, 1, BLK), lambda i: (i, 0, 0))


def kernel(x, edge_index, batch, W_rel1, W_root1, b1, W_rel2, W_root2, b2,
           gamma, beta, W_lin1, b_lin1, W_lin2, b_lin2):
  src2 = edge_index[0].reshape(32, TPC, CH)
  dst2 = edge_index[1].reshape(32, TPC, CH)
  idx = src2 | (dst2 << 16)
  zeros = jnp.zeros((RPT, D), jnp.float32)
  ones = jnp.ones((CH,), jnp.float32)

  xr = pl.pallas_call(
      _xr_body,
      grid=(NB,),
      in_specs=[_row_spec(), _full((D, D)), _full((1, D))],
      out_specs=_row_spec(),
      out_shape=jax.ShapeDtypeStruct((N, D), jnp.float32),
  )(x, W_root1, b1.reshape(1, D))

  aggp1, cntp = _sc_spmm_cnt(x, idx, zeros, ones)

  cnt0 = cntp[0, :N].reshape(NB, 1, BLK)
  cnt1 = cntp[1, :N].reshape(NB, 1, BLK)
  batch3 = batch.reshape(NB, 1, BLK)

  h = pl.pallas_call(
      _tc1_body,
      grid=(NB,),
      in_specs=[
          pl.BlockSpec((2, BLK, D), lambda i: (0, i, 0)),
          _vec_spec(), _vec_spec(),
          _row_spec(),
          _full((D, D)),
      ],
      out_specs=_row_spec(),
      out_shape=jax.ShapeDtypeStruct((N, D), jnp.float32),
  )(aggp1, cnt0, cnt1, xr, W_rel1)

  aggp2 = _sc_spmm(h, idx, zeros, ones)
  if isinstance(aggp2, (list, tuple)):
    aggp2 = aggp2[0]

  row23 = lambda p, i: (i * (1 - p) + (NB - 1) * p, 0)
  out = pl.pallas_call(
      _tc23_body,
      grid=(2, NB),
      in_specs=[
          pl.BlockSpec((2, BLK, D),
                       lambda p, i: (0, i * (1 - p) + (NB - 1) * p, 0)),
          pl.BlockSpec((1, 1, BLK),
                       lambda p, i: (i * (1 - p) + (NB - 1) * p, 0, 0)),
          pl.BlockSpec((1, 1, BLK),
                       lambda p, i: (i * (1 - p) + (NB - 1) * p, 0, 0)),
          pl.BlockSpec((BLK, D), row23),
          pl.BlockSpec((D, D), lambda p, i: (0, 0)),
          pl.BlockSpec((D, D), lambda p, i: (0, 0)),
          pl.BlockSpec((1, D), lambda p, i: (0, 0)),
          pl.BlockSpec((1, D), lambda p, i: (0, 0)),
          pl.BlockSpec((1, D), lambda p, i: (0, 0)),
          pl.BlockSpec((1, 1, BLK), lambda p, i: (i, 0, 0)),
          pl.BlockSpec((D, D), lambda p, i: (0, 0)),
          pl.BlockSpec((D, D), lambda p, i: (0, 0)),
          pl.BlockSpec((1, D), lambda p, i: (0, 0)),
          pl.BlockSpec((D, D), lambda p, i: (0, 0)),
          pl.BlockSpec((1, D), lambda p, i: (0, 0)),
      ],
      out_specs=pl.BlockSpec((G, D), lambda p, i: (0, 0)),
      out_shape=jax.ShapeDtypeStruct((G, D), jnp.float32),
      scratch_shapes=[pltpu.VMEM((N, D), jnp.float32),
                      pltpu.VMEM((8, D), jnp.float32),
                      pltpu.VMEM((G, D), jnp.float32),
                      pltpu.VMEM((G, D), jnp.float32),
                      pltpu.VMEM((G, 1), jnp.float32)],
  )(aggp2, cnt0, cnt1, h, W_rel2, W_root2, b2.reshape(1, D),
    gamma.reshape(1, D), beta.reshape(1, D), batch3,
    W_lin1[:D], W_lin1[D:], b_lin1.reshape(1, D), W_lin2,
    b_lin2.reshape(1, D))

  return out
